# Initial kernel scaffold; baseline (speedup 1.0000x reference)
import jax
import jax.numpy as jnp
from jax.experimental import pallas as pl


def kernel(x, mu, theta0, theta1, theta2, theta3, theta4, theta5, theta6, theta7, theta8):
    raise NotImplementedError("write your pallas kernel here")


# speedup vs baseline: 16.8484x; 16.8484x over previous
import functools

import jax
import jax.numpy as jnp
from jax.experimental import pallas as pl
from jax.experimental.pallas import tpu as pltpu

B, D, K = 1024, 512, 32
BT = 128
NTH = 511


def _circuit_kernel(xt_ref, mu_ref, th_ref, out_ref):
    xb = xt_ref[...][:, None, :]
    mk = mu_ref[...][:, :, None]
    diff = xb - mk
    cur = -0.5 * (diff * diff)

    off = 0
    f = D // 2
    while f >= 1:
        pair = cur.reshape(f, 2, K, BT)
        prod = pair[:, 0] + pair[:, 1]
        m = jnp.max(prod, axis=1, keepdims=True)
        e = jnp.exp(prod - m)
        th = th_ref[off:off + f]
        w = jax.nn.softmax(th, axis=-1)
        s = jax.lax.dot_general(
            w, e,
            dimension_numbers=(((2,), (1,)), ((0,), (0,))),
            preferred_element_type=jnp.float32,
        )
        cur = jnp.log(s) + m
        off += f
        f //= 2

    out_ref[:, :] = cur[0]


@jax.jit
def kernel(x, mu, theta0, theta1, theta2, theta3, theta4, theta5, theta6,
           theta7, theta8):
    thetas = [theta0, theta1, theta2, theta3, theta4, theta5, theta6,
              theta7, theta8]
    xt = jnp.transpose(x.reshape(B, D))
    mu2 = mu.reshape(D, K)
    th = jnp.concatenate([t.reshape(-1, K, K) for t in thetas], axis=0)

    grid = (B // BT,)
    out = pl.pallas_call(
        _circuit_kernel,
        grid=grid,
        in_specs=[
            pl.BlockSpec((D, BT), lambda i: (0, i)),
            pl.BlockSpec((D, K), lambda i: (0, 0)),
            pl.BlockSpec((NTH, K, K), lambda i: (0, 0, 0)),
        ],
        out_specs=pl.BlockSpec((K, BT), lambda i: (0, i)),
        out_shape=jax.ShapeDtypeStruct((K, B), jnp.float32),
        compiler_params=pltpu.CompilerParams(
            dimension_semantics=("parallel",),
        ),
    )(xt, mu2, th)
    return jnp.transpose(out).reshape(B, 1, K)


# speedup vs baseline: 20.7529x; 1.2317x over previous
import jax
import jax.numpy as jnp
from jax.experimental import pallas as pl
from jax.experimental.pallas import tpu as pltpu

B, D, K = 1024, 512, 32
BT = 128
NTH = 511


def _softmax_kernel(th_ref, w_ref):
    w_ref[...] = jax.nn.softmax(th_ref[...], axis=-1)


def _circuit_kernel(xt_ref, mu_ref, w_ref, out_ref):
    xb = xt_ref[...][:, None, :]
    mk = mu_ref[...][:, :, None]
    diff = xb - mk
    d2 = -0.5 * (diff * diff)

    f = D // 2
    pair = d2.reshape(f, 2, K, BT)
    prod = pair[:, 0] + pair[:, 1]
    m = jnp.max(prod, axis=1, keepdims=True)
    e = jnp.exp(prod - m)

    off = 0
    while True:
        w = w_ref[off:off + f]
        s = jax.lax.dot_general(
            w, e,
            dimension_numbers=(((2,), (1,)), ((0,), (0,))),
            preferred_element_type=jnp.float32,
        )
        if f == 1:
            out_ref[...] = m[0] + jnp.log(s[0])
            break
        t = jnp.max(s, axis=1, keepdims=True)
        m = m + jnp.log(t)
        off += f
        e = s * (1.0 / t)
        ep = e.reshape(f // 2, 2, K, BT)
        e = ep[:, 0] * ep[:, 1]
        mp = m.reshape(f // 2, 2, 1, BT)
        m = mp[:, 0] + mp[:, 1]
        f //= 2


@jax.jit
def kernel(x, mu, theta0, theta1, theta2, theta3, theta4, theta5, theta6,
           theta7, theta8):
    thetas = [theta0, theta1, theta2, theta3, theta4, theta5, theta6,
              theta7, theta8]
    xt = jnp.transpose(x.reshape(B, D))
    mu2 = mu.reshape(D, K)
    th = jnp.concatenate([t.reshape(-1, K, K) for t in thetas], axis=0)

    w = pl.pallas_call(
        _softmax_kernel,
        out_shape=jax.ShapeDtypeStruct((NTH, K, K), jnp.float32),
    )(th)

    out = pl.pallas_call(
        _circuit_kernel,
        grid=(B // BT,),
        in_specs=[
            pl.BlockSpec((D, BT), lambda i: (0, i)),
            pl.BlockSpec((D, K), lambda i: (0, 0)),
            pl.BlockSpec((NTH, K, K), lambda i: (0, 0, 0)),
        ],
        out_specs=pl.BlockSpec((K, BT), lambda i: (0, i)),
        out_shape=jax.ShapeDtypeStruct((K, B), jnp.float32),
        compiler_params=pltpu.CompilerParams(
            dimension_semantics=("parallel",),
        ),
    )(xt, mu2, w)
    return jnp.transpose(out).reshape(B, 1, K)


# speedup vs baseline: 22.3578x; 1.0773x over previous
import jax
import jax.numpy as jnp
from jax.experimental import pallas as pl
from jax.experimental.pallas import tpu as pltpu

B, D, K = 1024, 512, 32
BT = 128
NTH = 511
F0 = D // 2


def _prep_kernel(mu_ref, *refs):
    th_refs = refs[:9]
    w_ref, a_ref = refs[9:]
    off = 0
    for th_ref in th_refs:
        th = th_ref[...]
        f = th.shape[0]
        u = jnp.exp(th)
        ones = jnp.ones((f, K, 1), jnp.float32)
        z = jax.lax.dot_general(
            u, ones,
            dimension_numbers=(((2,), (1,)), ((0,), (0,))),
            preferred_element_type=jnp.float32,
        )
        w_ref[off:off + f] = u * (1.0 / z)
        off += f
    mu = mu_ref[...]
    musq = (mu * mu).reshape(F0, 2, K)
    a_ref[...] = -0.5 * (musq[:, 0] + musq[:, 1])


def _circuit_kernel(x_ref, mup_ref, w_ref, a_ref, out_ref):
    xt = jnp.transpose(x_ref[...])
    xsq = (xt * xt).reshape(F0, 2, BT)
    bv = -0.5 * (xsq[:, 0] + xsq[:, 1])
    xr = xt.reshape(F0, 2, BT)
    cc = jax.lax.dot_general(
        mup_ref[...], xr,
        dimension_numbers=(((1,), (1,)), ((0,), (0,))),
        preferred_element_type=jnp.float32,
    )
    prod = cc + a_ref[...][:, :, None]
    m = jnp.max(prod, axis=1, keepdims=True)
    e = jnp.exp(prod - m)
    m = m + bv[:, None, :]

    off = 0
    f = F0
    while True:
        w = w_ref[off:off + f]
        s = jax.lax.dot_general(
            w, e,
            dimension_numbers=(((2,), (1,)), ((0,), (0,))),
            preferred_element_type=jnp.float32,
        )
        if f == 1:
            out_ref[...] = jnp.transpose(m[0] + jnp.log(s[0]))
            break
        t = jnp.max(s, axis=1, keepdims=True)
        m = m + jnp.log(t)
        off += f
        e = s * (1.0 / t)
        ep = e.reshape(f // 2, 2, K, BT)
        e = ep[:, 0] * ep[:, 1]
        mp = m.reshape(f // 2, 2, 1, BT)
        m = mp[:, 0] + mp[:, 1]
        f //= 2


@jax.jit
def kernel(x, mu, theta0, theta1, theta2, theta3, theta4, theta5, theta6,
           theta7, theta8):
    thetas = [theta0, theta1, theta2, theta3, theta4, theta5, theta6,
              theta7, theta8]
    x2 = x.reshape(B, D)
    mu2 = mu.reshape(D, K)
    mup = mu2.reshape(F0, 2, K)

    w, a = pl.pallas_call(
        _prep_kernel,
        out_shape=[
            jax.ShapeDtypeStruct((NTH, K, K), jnp.float32),
            jax.ShapeDtypeStruct((F0, K), jnp.float32),
        ],
    )(mu2, *thetas)

    out = pl.pallas_call(
        _circuit_kernel,
        grid=(B // BT,),
        in_specs=[
            pl.BlockSpec((BT, D), lambda i: (i, 0)),
            pl.BlockSpec((F0, 2, K), lambda i: (0, 0, 0)),
            pl.BlockSpec((NTH, K, K), lambda i: (0, 0, 0)),
            pl.BlockSpec((F0, K), lambda i: (0, 0)),
        ],
        out_specs=pl.BlockSpec((BT, K), lambda i: (i, 0)),
        out_shape=jax.ShapeDtypeStruct((B, K), jnp.float32),
        compiler_params=pltpu.CompilerParams(
            dimension_semantics=("parallel",),
        ),
    )(x2, mup, w, a)
    return out.reshape(B, 1, K)


# speedup vs baseline: 25.3846x; 1.1354x over previous
import jax
import jax.numpy as jnp
from jax.experimental import pallas as pl
from jax.experimental.pallas import tpu as pltpu

B, D, K = 1024, 512, 32
BT = 256
NTH = 511
F0 = D // 2


def _prep_kernel(mu_ref, *refs):
    th_refs = refs[:9]
    w_ref, a_ref = refs[9:]
    off = 0
    for th_ref in th_refs:
        th = th_ref[...]
        f = th.shape[0]
        u = jnp.exp(th)
        ones = jnp.ones((f, K, 1), jnp.float32)
        z = jax.lax.dot_general(
            u, ones,
            dimension_numbers=(((2,), (1,)), ((0,), (0,))),
            preferred_element_type=jnp.float32,
        )
        w_ref[off:off + f] = u * (1.0 / z)
        off += f
    mu = mu_ref[...]
    musq = (mu * mu).reshape(F0, 2, K)
    a_ref[...] = -0.5 * (musq[:, 0] + musq[:, 1])


def _circuit_kernel(x_ref, mup_ref, w_ref, a_ref, out_ref):
    xt = jnp.transpose(x_ref[...])
    xsq = (xt * xt).reshape(F0, 2, BT)
    bv = -0.5 * (xsq[:, 0] + xsq[:, 1])
    xr = xt.reshape(F0, 2, BT)
    cc = jax.lax.dot_general(
        mup_ref[...], xr,
        dimension_numbers=(((1,), (1,)), ((0,), (0,))),
        preferred_element_type=jnp.float32,
    )
    prod = cc + a_ref[...][:, :, None]
    m = jnp.max(prod, axis=1, keepdims=True)
    e = jnp.exp(prod - m)
    m = m + bv[:, None, :]

    off = 0
    f = F0
    while True:
        w = w_ref[off:off + f]
        s = jax.lax.dot_general(
            w, e,
            dimension_numbers=(((2,), (1,)), ((0,), (0,))),
            preferred_element_type=jnp.float32,
        )
        if f == 1:
            out_ref[...] = jnp.transpose(m[0] + jnp.log(s[0]))
            break
        t = jnp.max(s, axis=1, keepdims=True)
        m = m + jnp.log(t)
        off += f
        e = s * (1.0 / t)
        ep = e.reshape(f // 2, 2, K, BT)
        e = ep[:, 0] * ep[:, 1]
        mp = m.reshape(f // 2, 2, 1, BT)
        m = mp[:, 0] + mp[:, 1]
        f //= 2


@jax.jit
def kernel(x, mu, theta0, theta1, theta2, theta3, theta4, theta5, theta6,
           theta7, theta8):
    thetas = [theta0, theta1, theta2, theta3, theta4, theta5, theta6,
              theta7, theta8]
    x2 = x.reshape(B, D)
    mu2 = mu.reshape(D, K)
    mup = mu2.reshape(F0, 2, K)

    w, a = pl.pallas_call(
        _prep_kernel,
        out_shape=[
            jax.ShapeDtypeStruct((NTH, K, K), jnp.float32),
            jax.ShapeDtypeStruct((F0, K), jnp.float32),
        ],
    )(mu2, *thetas)

    out = pl.pallas_call(
        _circuit_kernel,
        grid=(B // BT,),
        in_specs=[
            pl.BlockSpec((BT, D), lambda i: (i, 0)),
            pl.BlockSpec((F0, 2, K), lambda i: (0, 0, 0)),
            pl.BlockSpec((NTH, K, K), lambda i: (0, 0, 0)),
            pl.BlockSpec((F0, K), lambda i: (0, 0)),
        ],
        out_specs=pl.BlockSpec((BT, K), lambda i: (i, 0)),
        out_shape=jax.ShapeDtypeStruct((B, K), jnp.float32),
        compiler_params=pltpu.CompilerParams(
            dimension_semantics=("parallel",),
        ),
    )(x2, mup, w, a)
    return out.reshape(B, 1, K)


# speedup vs baseline: 26.8943x; 1.0595x over previous
import jax
import jax.numpy as jnp
from jax.experimental import pallas as pl
from jax.experimental.pallas import tpu as pltpu

B, D, K = 1024, 512, 32
BT = 512
NTH = 511
F0 = D // 2


def _prep_kernel(mu_ref, *refs):
    th_refs = refs[:9]
    w_ref, a_ref = refs[9:]
    off = 0
    for th_ref in th_refs:
        th = th_ref[...]
        f = th.shape[0]
        u = jnp.exp(th)
        ones = jnp.ones((f, K, 1), jnp.float32)
        z = jax.lax.dot_general(
            u, ones,
            dimension_numbers=(((2,), (1,)), ((0,), (0,))),
            preferred_element_type=jnp.float32,
        )
        w_ref[off:off + f] = u * (1.0 / z)
        off += f
    mu = mu_ref[...]
    musq = (mu * mu).reshape(F0, 2, K)
    a_ref[...] = -0.5 * (musq[:, 0] + musq[:, 1])


def _circuit_kernel(x_ref, mup_ref, w_ref, a_ref, out_ref):
    xt = jnp.transpose(x_ref[...])
    xsq = (xt * xt).reshape(F0, 2, BT)
    bv = -0.5 * (xsq[:, 0] + xsq[:, 1])
    xr = xt.reshape(F0, 2, BT)
    cc = jax.lax.dot_general(
        mup_ref[...], xr,
        dimension_numbers=(((1,), (1,)), ((0,), (0,))),
        preferred_element_type=jnp.float32,
    )
    prod = cc + a_ref[...][:, :, None]
    m = jnp.max(prod, axis=1, keepdims=True)
    e = jnp.exp(prod - m)
    m = m + bv[:, None, :]

    off = 0
    f = F0
    while True:
        w = w_ref[off:off + f]
        s = jax.lax.dot_general(
            w, e,
            dimension_numbers=(((2,), (1,)), ((0,), (0,))),
            preferred_element_type=jnp.float32,
        )
        if f == 1:
            out_ref[...] = jnp.transpose(m[0] + jnp.log(s[0]))
            break
        t = jnp.max(s, axis=1, keepdims=True)
        m = m + jnp.log(t)
        off += f
        e = s * (1.0 / t)
        ep = e.reshape(f // 2, 2, K, BT)
        e = ep[:, 0] * ep[:, 1]
        mp = m.reshape(f // 2, 2, 1, BT)
        m = mp[:, 0] + mp[:, 1]
        f //= 2


@jax.jit
def kernel(x, mu, theta0, theta1, theta2, theta3, theta4, theta5, theta6,
           theta7, theta8):
    thetas = [theta0, theta1, theta2, theta3, theta4, theta5, theta6,
              theta7, theta8]
    x2 = x.reshape(B, D)
    mu2 = mu.reshape(D, K)
    mup = mu2.reshape(F0, 2, K)

    w, a = pl.pallas_call(
        _prep_kernel,
        out_shape=[
            jax.ShapeDtypeStruct((NTH, K, K), jnp.float32),
            jax.ShapeDtypeStruct((F0, K), jnp.float32),
        ],
    )(mu2, *thetas)

    out = pl.pallas_call(
        _circuit_kernel,
        grid=(B // BT,),
        in_specs=[
            pl.BlockSpec((BT, D), lambda i: (i, 0)),
            pl.BlockSpec((F0, 2, K), lambda i: (0, 0, 0)),
            pl.BlockSpec((NTH, K, K), lambda i: (0, 0, 0)),
            pl.BlockSpec((F0, K), lambda i: (0, 0)),
        ],
        out_specs=pl.BlockSpec((BT, K), lambda i: (i, 0)),
        out_shape=jax.ShapeDtypeStruct((B, K), jnp.float32),
        compiler_params=pltpu.CompilerParams(
            dimension_semantics=("parallel",),
        ),
    )(x2, mup, w, a)
    return out.reshape(B, 1, K)


# speedup vs baseline: 29.5384x; 1.0983x over previous
import jax
import jax.numpy as jnp
from jax.experimental import pallas as pl
from jax.experimental.pallas import tpu as pltpu

B, D, K = 1024, 512, 32
BT = 512
NTH = 511
F0 = D // 2


def _circuit_kernel(x_ref, mu_ref, *refs):
    th_refs = refs[:9]
    out_ref, w_scr, a_scr = refs[9:]

    @pl.when(pl.program_id(0) == 0)
    def _prep():
        off = 0
        for th_ref in th_refs:
            th = th_ref[...]
            f = th.shape[0]
            u = jnp.exp(th)
            ones = jnp.ones((f, K, 1), jnp.float32)
            z = jax.lax.dot_general(
                u, ones,
                dimension_numbers=(((2,), (1,)), ((0,), (0,))),
                preferred_element_type=jnp.float32,
            )
            w_scr[off:off + f] = u * (1.0 / z)
            off += f
        mu = mu_ref[...]
        musq = (mu * mu).reshape(F0, 2, K)
        a_scr[...] = -0.5 * (musq[:, 0] + musq[:, 1])

    xt = jnp.transpose(x_ref[...])
    xsq = (xt * xt).reshape(F0, 2, BT)
    bv = -0.5 * (xsq[:, 0] + xsq[:, 1])
    xr = xt.reshape(F0, 2, BT)
    mup = mu_ref[...].reshape(F0, 2, K)
    cc = jax.lax.dot_general(
        mup, xr,
        dimension_numbers=(((1,), (1,)), ((0,), (0,))),
        preferred_element_type=jnp.float32,
    )
    prod = cc + a_scr[...][:, :, None]
    m = jnp.max(prod, axis=1, keepdims=True)
    e = jnp.exp(prod - m)
    m = m + bv[:, None, :]

    off = 0
    f = F0
    while True:
        w = w_scr[off:off + f]
        s = jax.lax.dot_general(
            w, e,
            dimension_numbers=(((2,), (1,)), ((0,), (0,))),
            preferred_element_type=jnp.float32,
        )
        if f == 1:
            out_ref[...] = jnp.transpose(m[0] + jnp.log(s[0]))
            break
        t = jnp.max(s, axis=1, keepdims=True)
        m = m + jnp.log(t)
        off += f
        e = s * (1.0 / t)
        ep = e.reshape(f // 2, 2, K, BT)
        e = ep[:, 0] * ep[:, 1]
        mp = m.reshape(f // 2, 2, 1, BT)
        m = mp[:, 0] + mp[:, 1]
        f //= 2


@jax.jit
def kernel(x, mu, theta0, theta1, theta2, theta3, theta4, theta5, theta6,
           theta7, theta8):
    thetas = [theta0, theta1, theta2, theta3, theta4, theta5, theta6,
              theta7, theta8]
    x2 = x.reshape(B, D)
    mu2 = mu.reshape(D, K)

    th_specs = [
        pl.BlockSpec(t.shape, lambda i: (0,) * t.ndim)
        for t in [jnp.zeros((max(F0 >> j, 1), K, K)) for j in range(9)]
    ]
    out = pl.pallas_call(
        _circuit_kernel,
        grid=(B // BT,),
        in_specs=[
            pl.BlockSpec((BT, D), lambda i: (i, 0)),
            pl.BlockSpec((D, K), lambda i: (0, 0)),
        ] + th_specs,
        out_specs=pl.BlockSpec((BT, K), lambda i: (i, 0)),
        out_shape=jax.ShapeDtypeStruct((B, K), jnp.float32),
        scratch_shapes=[
            pltpu.VMEM((NTH, K, K), jnp.float32),
            pltpu.VMEM((F0, K), jnp.float32),
        ],
        compiler_params=pltpu.CompilerParams(
            dimension_semantics=("arbitrary",),
        ),
    )(x2, mu2, *thetas)
    return out.reshape(B, 1, K)


# speedup vs baseline: 32.7275x; 1.1080x over previous
import jax
import jax.numpy as jnp
from jax.experimental import pallas as pl
from jax.experimental.pallas import tpu as pltpu

B, D, K = 1024, 512, 32
BT = 512
NTH = 511
F0 = D // 2


def _circuit_kernel(x_ref, mu_ref, *refs):
    th_refs = refs[:9]
    out_ref, w_scr, mup_scr, xb_scr = refs[9:]

    @pl.when(pl.program_id(0) == 0)
    def _prep():
        off = 0
        for th_ref in th_refs:
            th = th_ref[...]
            f = th.shape[0]
            u = jnp.exp(th)
            ones = jnp.ones((f, K, 1), jnp.float32)
            z = jax.lax.dot_general(
                u, ones,
                dimension_numbers=(((2,), (1,)), ((0,), (0,))),
                preferred_element_type=jnp.float32,
            )
            w_scr[off:off + f] = u * (1.0 / z)
            off += f
        mu = mu_ref[...].reshape(F0, 2, K)
        mup_scr[:, 0:2, :] = mu
        musq = mu * mu
        mup_scr[:, 2, :] = -0.5 * (musq[:, 0] + musq[:, 1])
        xb_scr[:, 2, :] = jnp.ones((F0, BT), jnp.float32)

    xt = jnp.transpose(x_ref[...])
    xsq = (xt * xt).reshape(F0, 2, BT)
    bv = -0.5 * (xsq[:, 0] + xsq[:, 1])
    xb_scr[:, 0:2, :] = xt.reshape(F0, 2, BT)
    prod = jax.lax.dot_general(
        mup_scr[...], xb_scr[...],
        dimension_numbers=(((1,), (1,)), ((0,), (0,))),
        preferred_element_type=jnp.float32,
    )
    m = jnp.max(prod, axis=1, keepdims=True)
    e = jnp.exp(prod - m)
    m = m + bv[:, None, :]

    off = 0
    f = F0
    while True:
        s = jax.lax.dot_general(
            w_scr[off:off + f], e,
            dimension_numbers=(((2,), (1,)), ((0,), (0,))),
            preferred_element_type=jnp.float32,
        )
        if f == 1:
            out_ref[...] = jnp.transpose(m[0] + jnp.log(s[0]))
            break
        off += f
        f //= 2
        sp = s.reshape(f, 2, K, BT)
        sp = sp[:, 0] * sp[:, 1]
        mp = m.reshape(f, 2, 1, BT)
        m = mp[:, 0] + mp[:, 1]
        ones = jnp.ones((f, 1, K), jnp.float32)
        t = jax.lax.dot_general(
            ones, sp,
            dimension_numbers=(((2,), (1,)), ((0,), (0,))),
            preferred_element_type=jnp.float32,
        )
        e = sp * (1.0 / t)
        m = m + jnp.log(t)


@jax.jit
def kernel(x, mu, theta0, theta1, theta2, theta3, theta4, theta5, theta6,
           theta7, theta8):
    thetas = [theta0, theta1, theta2, theta3, theta4, theta5, theta6,
              theta7, theta8]
    x2 = x.reshape(B, D)
    mu2 = mu.reshape(D, K)

    th_specs = [
        pl.BlockSpec((max(F0 >> j, 1), K, K), lambda i: (0, 0, 0))
        for j in range(9)
    ]
    out = pl.pallas_call(
        _circuit_kernel,
        grid=(B // BT,),
        in_specs=[
            pl.BlockSpec((BT, D), lambda i: (i, 0)),
            pl.BlockSpec((D, K), lambda i: (0, 0)),
        ] + th_specs,
        out_specs=pl.BlockSpec((BT, K), lambda i: (i, 0)),
        out_shape=jax.ShapeDtypeStruct((B, K), jnp.float32),
        scratch_shapes=[
            pltpu.VMEM((NTH, K, K), jnp.float32),
            pltpu.VMEM((F0, 3, K), jnp.float32),
            pltpu.VMEM((F0, 3, BT), jnp.float32),
        ],
        compiler_params=pltpu.CompilerParams(
            dimension_semantics=("arbitrary",),
        ),
    )(x2, mu2, *thetas)
    return out.reshape(B, 1, K)


# speedup vs baseline: 34.6913x; 1.0600x over previous
import jax
import jax.numpy as jnp
from jax.experimental import pallas as pl
from jax.experimental.pallas import tpu as pltpu

B, D, K = 1024, 512, 32
BT = 512
NTH = 511
F0 = D // 2


def _circuit_kernel(x_ref, mu_ref, *refs):
    th_refs = refs[:9]
    out_ref, w_scr, mup_scr, xb_scr = refs[9:]

    @pl.when(pl.program_id(0) == 0)
    def _prep():
        off = 0
        for th_ref in th_refs:
            th = th_ref[...]
            f = th.shape[0]
            u = jnp.exp(th)
            ones = jnp.ones((f, K, 1), jnp.float32)
            z = jax.lax.dot_general(
                u, ones,
                dimension_numbers=(((2,), (1,)), ((0,), (0,))),
                preferred_element_type=jnp.float32,
            )
            w_scr[off:off + f] = u * (1.0 / z)
            off += f
        mu = mu_ref[...].reshape(F0, 2, K)
        mup_scr[:, 0:2, :] = mu
        musq = mu * mu
        mup_scr[:, 2, :] = -0.5 * (musq[:, 0] + musq[:, 1])
        xb_scr[:, 2, :] = jnp.ones((F0, BT), jnp.float32)

    xt = jnp.transpose(x_ref[...].reshape(BT, D))
    xsq = (xt * xt).reshape(F0, 2, BT)
    bv = -0.5 * (xsq[:, 0] + xsq[:, 1])
    xb_scr[:, 0:2, :] = xt.reshape(F0, 2, BT)
    prod = jax.lax.dot_general(
        mup_scr[...], xb_scr[...],
        dimension_numbers=(((1,), (1,)), ((0,), (0,))),
        preferred_element_type=jnp.float32,
    )
    m = jnp.max(prod, axis=1, keepdims=True)
    e = jnp.exp(prod - m)
    m = m + bv[:, None, :]

    off = 0
    f = F0
    while True:
        s = jax.lax.dot_general(
            w_scr[off:off + f], e,
            dimension_numbers=(((2,), (1,)), ((0,), (0,))),
            preferred_element_type=jnp.float32,
        )
        if f == 1:
            res = jnp.transpose(m[0] + jnp.log(s[0]))
            out_ref[...] = res[:, None, :]
            break
        off += f
        f //= 2
        sp = s.reshape(f, 2, K, BT)
        sp = sp[:, 0] * sp[:, 1]
        mp = m.reshape(f, 2, 1, BT)
        m = mp[:, 0] + mp[:, 1]
        ones = jnp.ones((f, 1, K), jnp.float32)
        t = jax.lax.dot_general(
            ones, sp,
            dimension_numbers=(((2,), (1,)), ((0,), (0,))),
            preferred_element_type=jnp.float32,
        )
        e = sp * (1.0 / t)
        m = m + jnp.log(t)


@jax.jit
def kernel(x, mu, theta0, theta1, theta2, theta3, theta4, theta5, theta6,
           theta7, theta8):
    thetas = [theta0, theta1, theta2, theta3, theta4, theta5, theta6,
              theta7, theta8]

    th_specs = [
        pl.BlockSpec((max(F0 >> j, 1), K, K), lambda i: (0, 0, 0))
        for j in range(9)
    ]
    out = pl.pallas_call(
        _circuit_kernel,
        grid=(B // BT,),
        in_specs=[
            pl.BlockSpec((BT, 1, D), lambda i: (i, 0, 0)),
            pl.BlockSpec((D, 1, 1, K), lambda i: (0, 0, 0, 0)),
        ] + th_specs,
        out_specs=pl.BlockSpec((BT, 1, K), lambda i: (i, 0, 0)),
        out_shape=jax.ShapeDtypeStruct((B, 1, K), jnp.float32),
        scratch_shapes=[
            pltpu.VMEM((NTH, K, K), jnp.float32),
            pltpu.VMEM((F0, 3, K), jnp.float32),
            pltpu.VMEM((F0, 3, BT), jnp.float32),
        ],
        compiler_params=pltpu.CompilerParams(
            dimension_semantics=("arbitrary",),
        ),
    )(x, mu, *thetas)
    return out
